# speedup vs baseline: 3.5754x; 3.5754x over previous
import functools
import jax
import jax.numpy as jnp
import numpy as np
from jax.experimental import pallas as pl
from jax.experimental.pallas import tpu as pltpu

EMBED = 64
DM = 4096
OFF = (0, 16, 24, 48)
KP = 80
CLIP_HI = (12, 6, 23, 31)

ROWS_BLK = 512


def _proj_kernel(z_ref, w_ref, b_ref, p_ref):
    zw = jax.lax.dot_general(
        z_ref[...], w_ref[...], (((1,), (1,)), ((), ())),
        preferred_element_type=jnp.float32)
    p = zw * 0.5 + 0.125 * b_ref[...]
    p_ref[...] = p.astype(jnp.bfloat16)


def _mm_kernel(idx_ref, p_ref, out_ref):
    idx = idx_ref[...]
    col = jax.lax.broadcasted_iota(jnp.int32, (ROWS_BLK, KP), 1)
    mh = jnp.zeros((ROWS_BLK, KP), jnp.float32)
    for t in range(4):
        it = jnp.clip(idx[:, t], 0, CLIP_HI[t]) + OFF[t]
        mh = mh + jnp.where(col == it[:, None], 1.0, 0.0)
    out_ref[...] = jax.lax.dot_general(
        mh.astype(jnp.bfloat16), p_ref[...], (((1,), (0,)), ((), ())),
        preferred_element_type=jnp.float32)


def kernel(time_feats, month_w, weekday_w, hour_w, day_w, W, b):
    B, S, F = time_feats.shape
    N = B * S
    idx = time_feats.reshape(N, F).astype(jnp.int32)

    z = jnp.zeros((KP, 4 * EMBED), jnp.float32)
    for t, tbl in enumerate((month_w, weekday_w, hour_w, day_w)):
        z = jax.lax.dynamic_update_slice(z, tbl, (OFF[t], t * EMBED))

    p = pl.pallas_call(
        _proj_kernel,
        out_shape=jax.ShapeDtypeStruct((KP, DM), jnp.bfloat16),
    )(z, W, b.reshape(1, DM))

    nblk = N // ROWS_BLK
    out = pl.pallas_call(
        _mm_kernel,
        grid=(nblk,),
        in_specs=[
            pl.BlockSpec((ROWS_BLK, F), lambda i: (i, 0)),
            pl.BlockSpec((KP, DM), lambda i: (0, 0)),
        ],
        out_specs=pl.BlockSpec((ROWS_BLK, DM), lambda i: (i, 0)),
        out_shape=jax.ShapeDtypeStruct((N, DM), jnp.float32),
    )(idx, p)
    return out.reshape(B, S, DM)
